# baseline (speedup 1.0000x reference)
import functools

import jax
import jax.numpy as jnp
from jax import lax
from jax.experimental import pallas as pl
from jax.experimental.pallas import tpu as pltpu
from jax.experimental.pallas import tpu_sc as plsc

BATCH = 16384
EMBED_DIM = 32
PACK = 4
PACKED_W = PACK * EMBED_DIM
NUM_CORES = 2
NUM_SUBCORES = 16
NUM_WORKERS = NUM_CORES * NUM_SUBCORES
BPW = BATCH // NUM_WORKERS
CHUNK = 128
NCHUNKS = BPW // CHUNK
LANES = 16

_mesh = plsc.VectorSubcoreMesh(core_axis_name="c", subcore_axis_name="s")

_GATHER_DNUMS = lax.GatherDimensionNumbers(
    offset_dims=(), collapsed_slice_dims=(0,), start_index_map=(0,))


def _lane_sum(x, lane):
    for k in (1, 2, 4, 8):
        idx = (lane ^ k).reshape(LANES, 1)
        x = x + lax.gather(x, idx, _GATHER_DNUMS, (1,),
                           mode=lax.GatherScatterMode.PROMISE_IN_BOUNDS)
    return x


@functools.partial(
    pl.kernel,
    mesh=_mesh,
    out_type=jax.ShapeDtypeStruct((BATCH,), jnp.float32),
    scratch_types=[
        pltpu.VMEM((NCHUNKS, CHUNK), jnp.int32),
        pltpu.VMEM((NCHUNKS, CHUNK), jnp.int32),
        pltpu.VMEM((NCHUNKS, CHUNK), jnp.int32),
        pltpu.VMEM((NCHUNKS, CHUNK), jnp.int32),
        pltpu.VMEM((CHUNK, PACKED_W), jnp.float32),
        pltpu.VMEM((CHUNK, PACKED_W), jnp.float32),
        pltpu.VMEM((BPW,), jnp.float32),
        pltpu.SemaphoreType.DMA,
    ],
)
def _sc_dot(uid_hbm, iid_hbm, utab_hbm, itab_hbm, out_hbm,
            uid_v, iid_v, urow_v, irow_v, ubuf, ibuf, out_v, sem):
    wid = lax.axis_index("s") * NUM_CORES + lax.axis_index("c")
    base = wid * BPW

    for c in range(NCHUNKS):
        off = pl.ds(base + c * CHUNK, CHUNK)
        pltpu.sync_copy(uid_hbm.at[off], uid_v.at[c])
        pltpu.sync_copy(iid_hbm.at[off], iid_v.at[c])

    for c in range(NCHUNKS):
        for t in range(CHUNK // LANES):
            s = pl.ds(t * LANES, LANES)
            urow_v[c, s] = lax.shift_right_logical(uid_v[c, s], 2)
            irow_v[c, s] = lax.shift_right_logical(iid_v[c, s], 2)

    lane = lax.iota(jnp.int32, LANES)

    for c in range(NCHUNKS):
        cp_u = pltpu.async_copy(utab_hbm.at[urow_v.at[c]], ubuf, sem)
        cp_i = pltpu.async_copy(itab_hbm.at[irow_v.at[c]], ibuf, sem)
        cp_u.wait()
        cp_i.wait()

        def group_body(g, _, c=c):
            acc = jnp.zeros((LANES,), jnp.float32)
            gsl = pl.ds(g * LANES, LANES)
            uo_vec = (uid_v[c, gsl] & 3) << 5
            io_vec = (iid_v[c, gsl] & 3) << 5
            for j in range(LANES):
                r = g * LANES + j
                uo = uo_vec[j]
                io = io_vec[j]
                u0 = ubuf[r, pl.ds(uo, LANES)]
                u1 = ubuf[r, pl.ds(uo + LANES, LANES)]
                v0 = ibuf[r, pl.ds(io, LANES)]
                v1 = ibuf[r, pl.ds(io + LANES, LANES)]
                s = _lane_sum(u0 * v0 + u1 * v1, lane)
                acc = jnp.where(lane == j, s, acc)
            out_v[pl.ds(c * CHUNK + g * LANES, LANES)] = acc
            return 0

        lax.fori_loop(0, CHUNK // LANES, group_body, 0)

    pltpu.sync_copy(out_v, out_hbm.at[pl.ds(base, BPW)])


def kernel(user_ids, item_ids, user_table, item_table):
    ut = user_table.reshape(-1, PACKED_W)
    it = item_table.reshape(-1, PACKED_W)
    return _sc_dot(user_ids, item_ids, ut, it)


# speedup vs baseline: 3.0908x; 3.0908x over previous
import functools

import jax
import jax.numpy as jnp
from jax import lax
from jax.experimental import pallas as pl
from jax.experimental.pallas import tpu as pltpu
from jax.experimental.pallas import tpu_sc as plsc

BATCH = 16384
EMBED_DIM = 32
NUM_ROWS = 1000000
TILE_W = 128
TAIL_START = (NUM_ROWS // TILE_W) * TILE_W
LAST_BASE = TAIL_START - TILE_W
TAIL_N = NUM_ROWS - TAIL_START
NUM_CORES = 2
NUM_SUBCORES = 16
NUM_WORKERS = NUM_CORES * NUM_SUBCORES
BPW = BATCH // NUM_WORKERS
BSZ = 4
NBATCH = BPW // BSZ
LANES = 16
SLAB = BSZ * EMBED_DIM

_mesh = plsc.VectorSubcoreMesh(core_axis_name="c", subcore_axis_name="s")

_GATHER_DNUMS = lax.GatherDimensionNumbers(
    offset_dims=(), collapsed_slice_dims=(0,), start_index_map=(0,))


def _perm(x, idx_vec):
    return lax.gather(x, idx_vec.reshape(LANES, 1), _GATHER_DNUMS, (1,),
                      mode=lax.GatherScatterMode.PROMISE_IN_BOUNDS)


def _lane_sum(x, lane):
    for k in (1, 2, 4, 8):
        x = x + _perm(x, lane ^ k)
    return x


def _extract_col(buf, slot, c, lane):
    c16 = (c >> 4) << 4
    cl = jnp.full((LANES,), c & 15, jnp.int32)
    halves = []
    for h in range(2):
        acc = jnp.zeros((LANES,), jnp.float32)
        for d in range(LANES):
            row = buf[slot + h * LANES + d, pl.ds(c16, LANES)]
            acc = jnp.where(lane == d, _perm(row, cl), acc)
        halves.append(acc)
    return halves


@functools.partial(
    pl.kernel,
    mesh=_mesh,
    out_type=jax.ShapeDtypeStruct((BATCH,), jnp.float32),
    scratch_types=[
        pltpu.VMEM((BPW + LANES,), jnp.int32),
        pltpu.VMEM((BPW + LANES,), jnp.int32),
        pltpu.VMEM((2 * SLAB + EMBED_DIM, TILE_W), jnp.float32),
        pltpu.VMEM((2 * SLAB + EMBED_DIM, TILE_W), jnp.float32),
        pltpu.VMEM((TAIL_N * EMBED_DIM,), jnp.float32),
        pltpu.VMEM((TAIL_N * EMBED_DIM,), jnp.float32),
        pltpu.VMEM((BPW,), jnp.float32),
        pltpu.SemaphoreType.DMA,
    ],
)
def _sc_dot(uid_hbm, iid_hbm, utab_hbm, itab_hbm, utail_hbm, itail_hbm,
            out_hbm, uid_v, iid_v, ubuf, ibuf, utail1, itail1,
            out_v, sem):
    wid = lax.axis_index("s") * NUM_CORES + lax.axis_index("c")
    base = wid * BPW
    pltpu.sync_copy(uid_hbm.at[pl.ds(base, BPW)], uid_v.at[pl.ds(0, BPW)])
    pltpu.sync_copy(iid_hbm.at[pl.ds(base, BPW)], iid_v.at[pl.ds(0, BPW)])
    pltpu.sync_copy(utail_hbm, utail1)
    pltpu.sync_copy(itail_hbm, itail1)

    for d in range(EMBED_DIM):
        for t in range(TAIL_N // LANES):
            sl = pl.ds(t * LANES, LANES)
            ubuf[2 * SLAB + d, sl] = utail1[pl.ds(d * TAIL_N + t * LANES,
                                                  LANES)]
            ibuf[2 * SLAB + d, sl] = itail1[pl.ds(d * TAIL_N + t * LANES,
                                                  LANES)]

    lane = lax.iota(jnp.int32, LANES)

    def fire(g, p):
        uvec = uid_v[pl.ds(g * BSZ, LANES)]
        ivec = iid_v[pl.ds(g * BSZ, LANES)]
        cps = []
        for j in range(BSZ):
            ub = pl.multiple_of(
                jnp.minimum((uvec[j] >> 7) << 7, LAST_BASE), TILE_W)
            ib = pl.multiple_of(
                jnp.minimum((ivec[j] >> 7) << 7, LAST_BASE), TILE_W)
            slot = p * SLAB + j * EMBED_DIM
            cps.append(pltpu.async_copy(
                utab_hbm.at[:, pl.ds(ub, TILE_W)],
                ubuf.at[pl.ds(slot, EMBED_DIM), pl.ds(0, TILE_W)], sem))
            cps.append(pltpu.async_copy(
                itab_hbm.at[:, pl.ds(ib, TILE_W)],
                ibuf.at[pl.ds(slot, EMBED_DIM), pl.ds(0, TILE_W)], sem))
        return cps

    def compute(gc, p, acc):
        offs = jnp.maximum(gc, 0) * BSZ
        uvec = uid_v[pl.ds(offs, LANES)]
        ivec = iid_v[pl.ds(offs, LANES)]
        for j in range(BSZ):
            uidj = uvec[j]
            iidj = ivec[j]
            slot = p * SLAB + j * EMBED_DIM
            u_tail = uidj >= TAIL_START
            i_tail = iidj >= TAIL_START
            u_slot = jnp.where(u_tail, 2 * SLAB, slot)
            i_slot = jnp.where(i_tail, 2 * SLAB, slot)
            uc = jnp.where(
                u_tail, uidj - TAIL_START,
                uidj - jnp.minimum((uidj >> 7) << 7, LAST_BASE))
            ic = jnp.where(
                i_tail, iidj - TAIL_START,
                iidj - jnp.minimum((iidj >> 7) << 7, LAST_BASE))
            u0, u1 = _extract_col(ubuf, u_slot, uc, lane)
            v0, v1 = _extract_col(ibuf, i_slot, ic, lane)
            s = _lane_sum(u0 * v0 + u1 * v1, lane)
            lane_idx = lax.rem(gc, BSZ) * BSZ + j
            acc = jnp.where(lane == lane_idx, s, acc)
        return acc

    def body(g, acc):
        p = g & 1
        cps = fire(jnp.minimum(g, NBATCH - 1), p)
        gc = g - 1
        acc = compute(gc, 1 - p, acc)

        @pl.when((gc >= 0) & (lax.rem(gc, BSZ) == BSZ - 1))
        def _():
            out_v[pl.ds((gc // BSZ) * LANES, LANES)] = acc

        acc = jnp.where(lax.rem(gc, BSZ) == BSZ - 1,
                        jnp.zeros((LANES,), jnp.float32), acc)
        for cp in cps:
            cp.wait()
        return acc

    acc = lax.fori_loop(0, NBATCH + 1, body,
                        jnp.zeros((LANES,), jnp.float32))
    del acc

    pltpu.sync_copy(out_v, out_hbm.at[pl.ds(base, BPW)])


def kernel(user_ids, item_ids, user_table, item_table):
    utail = user_table[TAIL_START:, :].T.reshape(-1)
    itail = item_table[TAIL_START:, :].T.reshape(-1)
    return _sc_dot(user_ids, item_ids, user_table.T, item_table.T,
                   utail, itail)


# speedup vs baseline: 3.9601x; 1.2813x over previous
import functools

import jax
import jax.numpy as jnp
from jax import lax
from jax.experimental import pallas as pl
from jax.experimental.pallas import tpu as pltpu
from jax.experimental.pallas import tpu_sc as plsc

BATCH = 16384
SC_BATCH = 9728
TC_BATCH = BATCH - SC_BATCH
EMBED_DIM = 32
NUM_ROWS = 1000000
TILE_W = 128
TAIL_START = (NUM_ROWS // TILE_W) * TILE_W
LAST_BASE = TAIL_START - TILE_W
TAIL_N = NUM_ROWS - TAIL_START
NUM_CORES = 2
NUM_SUBCORES = 16
NUM_WORKERS = NUM_CORES * NUM_SUBCORES
BPW = SC_BATCH // NUM_WORKERS
BSZ = 4
NBATCH = BPW // BSZ
LANES = 16
SLAB = BSZ * EMBED_DIM
TCB = 128
TC_STEPS = TC_BATCH // TCB

_mesh = plsc.VectorSubcoreMesh(core_axis_name="c", subcore_axis_name="s")

_GATHER_DNUMS = lax.GatherDimensionNumbers(
    offset_dims=(), collapsed_slice_dims=(0,), start_index_map=(0,))


def _perm(x, idx_vec):
    return lax.gather(x, idx_vec.reshape(LANES, 1), _GATHER_DNUMS, (1,),
                      mode=lax.GatherScatterMode.PROMISE_IN_BOUNDS)


def _lane_sum(x, lane):
    for k in (1, 2, 4, 8):
        x = x + _perm(x, lane ^ k)
    return x


def _extract_col(buf, slot, c, lane):
    c16 = (c >> 4) << 4
    cl = jnp.full((LANES,), c & 15, jnp.int32)
    halves = []
    for h in range(2):
        acc = jnp.zeros((LANES,), jnp.float32)
        for d in range(LANES):
            row = buf[slot + h * LANES + d, pl.ds(c16, LANES)]
            acc = jnp.where(lane == d, _perm(row, cl), acc)
        halves.append(acc)
    return halves


@functools.partial(
    pl.kernel,
    mesh=_mesh,
    out_type=jax.ShapeDtypeStruct((SC_BATCH,), jnp.float32),
    scratch_types=[
        pltpu.VMEM((BPW + LANES,), jnp.int32),
        pltpu.VMEM((BPW + LANES,), jnp.int32),
        pltpu.VMEM((2 * SLAB + EMBED_DIM, TILE_W), jnp.float32),
        pltpu.VMEM((2 * SLAB + EMBED_DIM, TILE_W), jnp.float32),
        pltpu.VMEM((TAIL_N * EMBED_DIM,), jnp.float32),
        pltpu.VMEM((TAIL_N * EMBED_DIM,), jnp.float32),
        pltpu.VMEM((BPW,), jnp.float32),
        pltpu.SemaphoreType.DMA,
    ],
)
def _sc_dot(uid_hbm, iid_hbm, utab_hbm, itab_hbm, utail_hbm, itail_hbm,
            out_hbm, uid_v, iid_v, ubuf, ibuf, utail1, itail1,
            out_v, sem):
    wid = lax.axis_index("s") * NUM_CORES + lax.axis_index("c")
    base = wid * BPW
    pltpu.sync_copy(uid_hbm.at[pl.ds(base, BPW)], uid_v.at[pl.ds(0, BPW)])
    pltpu.sync_copy(iid_hbm.at[pl.ds(base, BPW)], iid_v.at[pl.ds(0, BPW)])
    pltpu.sync_copy(utail_hbm, utail1)
    pltpu.sync_copy(itail_hbm, itail1)

    for d in range(EMBED_DIM):
        for t in range(TAIL_N // LANES):
            sl = pl.ds(t * LANES, LANES)
            ubuf[2 * SLAB + d, sl] = utail1[pl.ds(d * TAIL_N + t * LANES,
                                                  LANES)]
            ibuf[2 * SLAB + d, sl] = itail1[pl.ds(d * TAIL_N + t * LANES,
                                                  LANES)]

    lane = lax.iota(jnp.int32, LANES)

    def fire(g, p):
        uvec = uid_v[pl.ds(g * BSZ, LANES)]
        ivec = iid_v[pl.ds(g * BSZ, LANES)]
        cps = []
        for j in range(BSZ):
            ub = pl.multiple_of(
                jnp.minimum((uvec[j] >> 7) << 7, LAST_BASE), TILE_W)
            ib = pl.multiple_of(
                jnp.minimum((ivec[j] >> 7) << 7, LAST_BASE), TILE_W)
            slot = p * SLAB + j * EMBED_DIM
            cps.append(pltpu.async_copy(
                utab_hbm.at[:, pl.ds(ub, TILE_W)],
                ubuf.at[pl.ds(slot, EMBED_DIM), pl.ds(0, TILE_W)], sem))
            cps.append(pltpu.async_copy(
                itab_hbm.at[:, pl.ds(ib, TILE_W)],
                ibuf.at[pl.ds(slot, EMBED_DIM), pl.ds(0, TILE_W)], sem))
        return cps

    def compute(gc, p, acc):
        offs = jnp.maximum(gc, 0) * BSZ
        uvec = uid_v[pl.ds(offs, LANES)]
        ivec = iid_v[pl.ds(offs, LANES)]
        for j in range(BSZ):
            uidj = uvec[j]
            iidj = ivec[j]
            slot = p * SLAB + j * EMBED_DIM
            u_tail = uidj >= TAIL_START
            i_tail = iidj >= TAIL_START
            u_slot = jnp.where(u_tail, 2 * SLAB, slot)
            i_slot = jnp.where(i_tail, 2 * SLAB, slot)
            uc = jnp.where(
                u_tail, uidj - TAIL_START,
                uidj - jnp.minimum((uidj >> 7) << 7, LAST_BASE))
            ic = jnp.where(
                i_tail, iidj - TAIL_START,
                iidj - jnp.minimum((iidj >> 7) << 7, LAST_BASE))
            u0, u1 = _extract_col(ubuf, u_slot, uc, lane)
            v0, v1 = _extract_col(ibuf, i_slot, ic, lane)
            s = _lane_sum(u0 * v0 + u1 * v1, lane)
            lane_idx = lax.rem(gc, BSZ) * BSZ + j
            acc = jnp.where(lane == lane_idx, s, acc)
        return acc

    def body(g, acc):
        p = g & 1
        cps = fire(jnp.minimum(g, NBATCH - 1), p)
        gc = g - 1
        acc = compute(gc, 1 - p, acc)

        @pl.when((gc >= 0) & (lax.rem(gc, BSZ) == BSZ - 1))
        def _():
            out_v[pl.ds((gc // BSZ) * LANES, LANES)] = acc

        acc = jnp.where(lax.rem(gc, BSZ) == BSZ - 1,
                        jnp.zeros((LANES,), jnp.float32), acc)
        for cp in cps:
            cp.wait()
        return acc

    acc = lax.fori_loop(0, NBATCH + 1, body,
                        jnp.zeros((LANES,), jnp.float32))
    del acc

    pltpu.sync_copy(out_v, out_hbm.at[pl.ds(base, BPW)])


def _tc_body(uid_sm, iid_sm, uid_ref, iid_ref, utab, itab, utail_ref,
             itail_ref, out_ref, ubuf, ibuf, sems):
    step = pl.program_id(0)

    def fire(bi, slot):
        for j in range(TCB):
            uidj = uid_sm[bi * TCB + j]
            iidj = iid_sm[bi * TCB + j]
            ub = pl.multiple_of(
                jnp.minimum((uidj >> 7) << 7, LAST_BASE), TILE_W)
            ib = pl.multiple_of(
                jnp.minimum((iidj >> 7) << 7, LAST_BASE), TILE_W)
            pltpu.make_async_copy(
                utab.at[:, pl.ds(ub, TILE_W)], ubuf.at[slot, j],
                sems.at[slot]).start()
            pltpu.make_async_copy(
                itab.at[:, pl.ds(ib, TILE_W)], ibuf.at[slot, j],
                sems.at[slot]).start()

    @pl.when(step == 0)
    def _():
        fire(0, 0)

    @pl.when(step < TC_STEPS - 1)
    def _():
        fire(step + 1, (step + 1) & 1)

    slot = step & 1
    for j in range(TCB):
        pltpu.make_async_copy(
            utab.at[:, pl.ds(0, TILE_W)], ubuf.at[slot, j],
            sems.at[slot]).wait()
        pltpu.make_async_copy(
            itab.at[:, pl.ds(0, TILE_W)], ibuf.at[slot, j],
            sems.at[slot]).wait()

    uids = uid_ref[...]
    iids = iid_ref[...]
    ucol = jnp.minimum(
        jnp.where(uids >= TAIL_START, uids - TAIL_START,
                  uids - jnp.minimum((uids >> 7) << 7, LAST_BASE)),
        TILE_W - 1)
    icol = jnp.minimum(
        jnp.where(iids >= TAIL_START, iids - TAIL_START,
                  iids - jnp.minimum((iids >> 7) << 7, LAST_BASE)),
        TILE_W - 1)
    colio = lax.broadcasted_iota(jnp.int32, (TCB, TILE_W), 1)
    uoh = (colio == lax.broadcast_in_dim(
        ucol, (TCB, TILE_W), (0, 1))).astype(jnp.float32)
    ioh = (colio == lax.broadcast_in_dim(
        icol, (TCB, TILE_W), (0, 1))).astype(jnp.float32)
    ue = lax.dot_general(ubuf[slot], uoh,
                         (((2,), (1,)), ((0,), (0,))),
                         preferred_element_type=jnp.float32)
    ie = lax.dot_general(ibuf[slot], ioh,
                         (((2,), (1,)), ((0,), (0,))),
                         preferred_element_type=jnp.float32)

    tailio = lax.broadcasted_iota(jnp.int32, (TCB, TAIL_N), 1)
    utoh = (tailio == lax.broadcast_in_dim(
        jnp.clip(uids - TAIL_START, 0, TAIL_N - 1),
        (TCB, TAIL_N), (0, 1))).astype(jnp.float32)
    itoh = (tailio == lax.broadcast_in_dim(
        jnp.clip(iids - TAIL_START, 0, TAIL_N - 1),
        (TCB, TAIL_N), (0, 1))).astype(jnp.float32)
    ut = jnp.dot(utoh, utail_ref[...],
                 preferred_element_type=jnp.float32)
    it = jnp.dot(itoh, itail_ref[...],
                 preferred_element_type=jnp.float32)
    um = lax.broadcast_in_dim(uids >= TAIL_START, (TCB, EMBED_DIM), (0, 1))
    im = lax.broadcast_in_dim(iids >= TAIL_START, (TCB, EMBED_DIM), (0, 1))
    ue = jnp.where(um, ut, ue)
    ie = jnp.where(im, it, ie)

    out_ref[...] = jnp.sum(ue * ie, axis=1, keepdims=True)


_tc_dot = pl.pallas_call(
    _tc_body,
    grid_spec=pltpu.PrefetchScalarGridSpec(
        num_scalar_prefetch=2,
        grid=(TC_STEPS,),
        in_specs=[
            pl.BlockSpec((TCB, 1), lambda i, u, v: (i, 0)),
            pl.BlockSpec((TCB, 1), lambda i, u, v: (i, 0)),
            pl.BlockSpec(memory_space=pl.ANY),
            pl.BlockSpec(memory_space=pl.ANY),
            pl.BlockSpec((TAIL_N, EMBED_DIM), lambda i, u, v: (0, 0)),
            pl.BlockSpec((TAIL_N, EMBED_DIM), lambda i, u, v: (0, 0)),
        ],
        out_specs=pl.BlockSpec((TCB, 1), lambda i, u, v: (i, 0)),
        scratch_shapes=[
            pltpu.VMEM((2, TCB, EMBED_DIM, TILE_W), jnp.float32),
            pltpu.VMEM((2, TCB, EMBED_DIM, TILE_W), jnp.float32),
            pltpu.SemaphoreType.DMA((2,)),
        ],
    ),
    out_shape=jax.ShapeDtypeStruct((TC_BATCH, 1), jnp.float32),
)


def kernel(user_ids, item_ids, user_table, item_table):
    utail1 = user_table[TAIL_START:, :].T.reshape(-1)
    itail1 = item_table[TAIL_START:, :].T.reshape(-1)
    utail2 = user_table[TAIL_START:, :]
    itail2 = item_table[TAIL_START:, :]
    out_sc = _sc_dot(user_ids[:SC_BATCH], item_ids[:SC_BATCH],
                     user_table.T, item_table.T, utail1, itail1)
    out_tc = _tc_dot(user_ids[SC_BATCH:], item_ids[SC_BATCH:],
                     user_ids[SC_BATCH:].reshape(TC_BATCH, 1),
                     item_ids[SC_BATCH:].reshape(TC_BATCH, 1),
                     user_table.T, item_table.T, utail2, itail2)
    return jnp.concatenate([out_sc, out_tc.reshape(TC_BATCH)])
